# baseline (speedup 1.0000x reference)
import functools

import numpy as np
import jax
import jax.numpy as jnp
from jax import lax
from jax.experimental import pallas as pl
from jax.experimental.pallas import tpu as pltpu
from jax.experimental.pallas import tpu_sc as plsc

_N = 100000
_D = 16
_NC = 2
_NS = 16
_NW = _NC * _NS
_SUB = 128
_RD = 4
_KS = 3
_RPS = 6272
_NACC = _RPS * _NS
_ROWS = _NACC * _D // 128

_SC_PARAMS = pltpu.CompilerParams(use_tc_tiling_on_sc=False)


def _pad_epw(e):
    step = _RD * _KS * _SUB
    return max(2 * step, ((e + _NW * step - 1) // (_NW * step)) * step)


def _count_k(epw):
    for k in (8, 6, 4, 2, 1):
        rows = epw // _SUB
        if rows % k == 0 and (rows // k) % _RD == 0:
            return k
    raise ValueError(epw)


def _sum_kernel(epw):
    k = _KS
    nchunk = epw // (k * _SUB)
    assert nchunk % _RD == 0 and nchunk >= 2 * _RD
    mesh = plsc.VectorSubcoreMesh(core_axis_name="c", subcore_axis_name="s")

    def body(x_h, ei_h, z2_h, out_h, acc, src_v, dst_v, rows_v,
             isem, gsem, ssem):
        c = lax.axis_index("c")
        s = lax.axis_index("s")
        w = c * _NS + s
        r0 = s * _RPS
        wrow = w * (epw // _SUB)

        def fire_idx(g, b):
            pltpu.async_copy(ei_h.at[0, pl.ds(wrow + g * k, k)], src_v[b],
                             isem[b])
            pltpu.async_copy(ei_h.at[1, pl.ds(wrow + g * k, k)], dst_v[b],
                             isem[b])

        def wait_idx(b):
            pltpu.make_async_copy(ei_h.at[0, pl.ds(0, k)], src_v[b],
                                  isem[b]).wait()
            pltpu.make_async_copy(ei_h.at[0, pl.ds(0, k)], dst_v[b],
                                  isem[b]).wait()

        def fire_gathers(b):
            for j in range(k):
                pltpu.async_copy(x_h.at[src_v[b].at[j]], rows_v[b].at[j],
                                 gsem[b])

        def wait_gathers(b):
            for j in range(k):
                pltpu.make_async_copy(x_h.at[pl.ds(0, _SUB)],
                                      rows_v[b].at[j], gsem[b]).wait()

        def fire_scatters(b):
            for j in range(k):
                pltpu.async_copy(rows_v[b].at[j], acc.at[dst_v[b].at[j]],
                                 ssem[b], add=True)

        def wait_scatters(b):
            for j in range(k):
                pltpu.make_async_copy(rows_v[b].at[j],
                                      acc.at[pl.ds(0, _SUB)], ssem[b]).wait()

        pltpu.sync_copy(z2_h.at[pl.ds(r0, _RPS)], acc.at[pl.ds(r0, _RPS)])
        plsc.subcore_barrier()

        fire_idx(0, 0)
        fire_idx(1, 1)
        fire_idx(2, 2)
        wait_idx(0)
        fire_gathers(0)
        fire_idx(3, 3)
        wait_idx(1)
        fire_gathers(1)
        wait_gathers(0)
        fire_scatters(0)

        def blk(i0, carry):
            for r in range(_RD):
                g = 2 + i0 * _RD + r
                wait_scatters(r)
                fire_idx(g + 2, r)
                wait_idx((2 + r) % _RD)
                fire_gathers((2 + r) % _RD)
                wait_gathers((1 + r) % _RD)
                fire_scatters((1 + r) % _RD)
            return carry

        lax.fori_loop(0, (nchunk - 4) // _RD, blk, 0)

        wait_scatters(0)
        wait_idx(2)
        fire_gathers(2)
        wait_gathers(1)
        fire_scatters(1)
        wait_scatters(1)
        wait_idx(3)
        fire_gathers(3)
        wait_gathers(2)
        fire_scatters(2)
        wait_gathers(3)
        fire_scatters(3)
        wait_scatters(2)
        wait_scatters(3)

        plsc.subcore_barrier()
        pltpu.sync_copy(acc.at[pl.ds(r0, _RPS)], out_h.at[c, pl.ds(r0, _RPS)])

    return pl.kernel(
        body,
        out_type=jax.ShapeDtypeStruct((_NC, _NACC, _D), jnp.float32),
        mesh=mesh,
        scratch_types=[
            pltpu.VMEM_SHARED((_NACC, _D), jnp.float32),
            [pltpu.VMEM((k, _SUB), jnp.int32) for _ in range(_RD)],
            [pltpu.VMEM((k, _SUB), jnp.int32) for _ in range(_RD)],
            [pltpu.VMEM((k, _SUB, _D), jnp.float32) for _ in range(_RD)],
            [pltpu.SemaphoreType.DMA for _ in range(_RD)],
            [pltpu.SemaphoreType.DMA for _ in range(_RD)],
            [pltpu.SemaphoreType.DMA for _ in range(_RD)],
        ],
        compiler_params=_SC_PARAMS,
    )


def _count_kernel(epw):
    k = _count_k(epw)
    nchunk = epw // (k * _SUB)
    assert nchunk % _RD == 0 and nchunk >= 2 * _RD
    mesh = plsc.VectorSubcoreMesh(core_axis_name="c", subcore_axis_name="s")

    def body(ei_h, z1_h, ones_h, cnt_h, cacc, dst_v, ones_v, isem, ssem):
        c = lax.axis_index("c")
        s = lax.axis_index("s")
        w = c * _NS + s
        r0 = s * _RPS
        wrow = w * (epw // _SUB)

        def fire_idx(g, b):
            pltpu.async_copy(ei_h.at[1, pl.ds(wrow + g * k, k)], dst_v[b],
                             isem[b])

        def wait_idx(b):
            pltpu.make_async_copy(ei_h.at[1, pl.ds(0, k)], dst_v[b],
                                  isem[b]).wait()

        def fire_scatters(b):
            for j in range(k):
                pltpu.async_copy(ones_v.at[0], cacc.at[dst_v[b].at[j]],
                                 ssem[b], add=True)

        def wait_scatters(b):
            for j in range(k):
                pltpu.make_async_copy(ones_v.at[0], cacc.at[pl.ds(0, _SUB)],
                                      ssem[b]).wait()

        pltpu.sync_copy(z1_h.at[pl.ds(r0, _RPS)], cacc.at[pl.ds(r0, _RPS)])
        pltpu.sync_copy(ones_h, ones_v)
        plsc.subcore_barrier()

        fire_idx(0, 0)
        fire_idx(1, 1)
        fire_idx(2, 2)
        wait_idx(0)
        fire_scatters(0)
        fire_idx(3, 3)
        wait_idx(1)
        fire_scatters(1)

        def blk(i0, carry):
            for r in range(_RD):
                g = 2 + i0 * _RD + r
                wait_scatters(r)
                fire_idx(g + 2, r)
                wait_idx((2 + r) % _RD)
                fire_scatters((2 + r) % _RD)
            return carry

        lax.fori_loop(0, (nchunk - 4) // _RD, blk, 0)

        wait_scatters(0)
        wait_idx(2)
        fire_scatters(2)
        wait_scatters(1)
        wait_idx(3)
        fire_scatters(3)
        wait_scatters(2)
        wait_scatters(3)

        plsc.subcore_barrier()
        pltpu.sync_copy(cacc.at[pl.ds(r0, _RPS)], cnt_h.at[c, pl.ds(r0, _RPS)])

    return pl.kernel(
        body,
        out_type=jax.ShapeDtypeStruct((_NC, _NACC), jnp.float32),
        mesh=mesh,
        scratch_types=[
            pltpu.VMEM_SHARED((_NACC,), jnp.float32),
            [pltpu.VMEM((k, _SUB), jnp.int32) for _ in range(_RD)],
            pltpu.VMEM((1, _SUB), jnp.float32),
            [pltpu.SemaphoreType.DMA for _ in range(_RD)],
            [pltpu.SemaphoreType.DMA for _ in range(_RD)],
        ],
        compiler_params=_SC_PARAMS,
    )


def _dense_body(p_ref, x_ref, c_ref, wl_ref, wr_ref, b_ref, o_ref):
    hi = jax.lax.Precision.HIGHEST
    cnt = jnp.maximum(c_ref[0] + c_ref[1], 1.0)
    agg = (p_ref[0] + p_ref[1]) / cnt
    o_ref[...] = (
        jnp.dot(agg, wl_ref[...], precision=hi, preferred_element_type=jnp.float32)
        + jnp.dot(x_ref[...], wr_ref[...], precision=hi,
                  preferred_element_type=jnp.float32)
        + b_ref[...]
    )


_BRD = 5000


def _dense(p, xr, cnt, wl, wr2, bias):
    grid = (_N // _BRD,)
    return pl.pallas_call(
        _dense_body,
        grid=grid,
        in_specs=[
            pl.BlockSpec((_NC, _BRD, _D), lambda i: (0, i, 0)),
            pl.BlockSpec((_BRD, _D), lambda i: (i, 0)),
            pl.BlockSpec((_NC, _BRD, 1), lambda i: (0, i, 0)),
            pl.BlockSpec((_D, _D), lambda i: (0, 0)),
            pl.BlockSpec((_D, _D), lambda i: (0, 0)),
            pl.BlockSpec((1, _D), lambda i: (0, 0)),
        ],
        out_specs=pl.BlockSpec((_BRD, _D), lambda i: (i, 0)),
        out_shape=jax.ShapeDtypeStruct((_N, _D), jnp.float32),
    )(p, xr, cnt, wl, wr2, bias)


def kernel(x, edge_index, Wl0, bl0, Wr0, Wlin0, blin0, Wl1, bl1, Wr1, Wlin1,
           blin1, Wl2, bl2, Wr2, Wlin2, blin2):
    ei = edge_index.astype(jnp.int32)
    e = ei.shape[1]
    epw = _pad_epw(e)
    pad = jnp.broadcast_to(jnp.array([[0], [_N]], jnp.int32),
                           (2, epw * _NW - e))
    ei2 = jnp.concatenate([ei, pad], axis=1).reshape(2, -1, _SUB)

    z2 = jnp.zeros((_NACC, _D), jnp.float32)
    z1 = jnp.zeros((_NACC,), jnp.float32)
    ones = jnp.ones((1, _SUB), jnp.float32)

    params = [(Wl0, bl0, Wr0, Wlin0, blin0), (Wl1, bl1, Wr1, Wlin1, blin1),
              (Wl2, bl2, Wr2, Wlin2, blin2)]

    sc_sum = _sum_kernel(epw)
    cnt = _count_kernel(epw)(ei2, z1, ones).reshape(_NC, _NACC, 1)

    xl = x
    for (Wl, bl, Wr, Wlin, blin) in params:
        p = sc_sum(xl, ei2, z2)
        xl = _dense(p, xl, cnt, Wl, Wr + Wlin, (bl + blin).reshape(1, _D))
    return xl


# speedup vs baseline: 1.5338x; 1.5338x over previous
import functools

import numpy as np
import jax
import jax.numpy as jnp
from jax import lax
from jax.experimental import pallas as pl
from jax.experimental.pallas import tpu as pltpu
from jax.experimental.pallas import tpu_sc as plsc

_N = 100000
_D = 16
_NC = 2
_NS = 16
_NW = _NC * _NS
_SUB = 128
_RD = 4
_KS = 3
_KC = 6
_RPS = 6272
_NACC = _RPS * _NS
_ROWS = _NACC * _D // 128

_FRAC0 = 0.655
_QUANT = 3072

_SC_PARAMS = pltpu.CompilerParams(use_tc_tiling_on_sc=False)


def _split_epw(e):
    epw0 = max(2 * _QUANT, -(-int(_FRAC0 * e / _NS) // _QUANT) * _QUANT)
    rest = max(0, e - _NS * epw0)
    epw1 = max(2 * _QUANT, -(-rest // (_NS * _QUANT)) * _QUANT)
    return epw0, epw1


def _sum_kernel(epw0, epw1):
    k = _KS
    n0 = epw0 // (k * _SUB)
    n1 = epw1 // (k * _SUB)
    assert n0 % _RD == 0 and n1 % _RD == 0 and min(n0, n1) >= 2 * _RD
    mesh = plsc.VectorSubcoreMesh(core_axis_name="c", subcore_axis_name="s")

    def body(x_h, ei_h, z2_h, out_h, acc, src_v, dst_v, rows_v,
             isem, gsem, ssem):
        c = lax.axis_index("c")
        s = lax.axis_index("s")
        r0 = s * _RPS
        rpw = jnp.where(c == 0, epw0 // _SUB, epw1 // _SUB)
        wrow = c * (_NS * (epw0 // _SUB)) + s * rpw
        nblk = jnp.where(c == 0, (n0 - 4) // _RD, (n1 - 4) // _RD)

        def fire_idx(g, b):
            pltpu.async_copy(ei_h.at[0, pl.ds(wrow + g * k, k)], src_v[b],
                             isem[b])
            pltpu.async_copy(ei_h.at[1, pl.ds(wrow + g * k, k)], dst_v[b],
                             isem[b])

        def wait_idx(b):
            pltpu.make_async_copy(ei_h.at[0, pl.ds(0, k)], src_v[b],
                                  isem[b]).wait()
            pltpu.make_async_copy(ei_h.at[0, pl.ds(0, k)], dst_v[b],
                                  isem[b]).wait()

        def fire_gathers(b):
            for j in range(k):
                pltpu.async_copy(x_h.at[src_v[b].at[j]], rows_v[b].at[j],
                                 gsem[b])

        def wait_gathers(b):
            for j in range(k):
                pltpu.make_async_copy(x_h.at[pl.ds(0, _SUB)],
                                      rows_v[b].at[j], gsem[b]).wait()

        def fire_scatters(b):
            for j in range(k):
                pltpu.async_copy(rows_v[b].at[j], acc.at[dst_v[b].at[j]],
                                 ssem[b], add=True)

        def wait_scatters(b):
            for j in range(k):
                pltpu.make_async_copy(rows_v[b].at[j],
                                      acc.at[pl.ds(0, _SUB)], ssem[b]).wait()

        pltpu.sync_copy(z2_h.at[pl.ds(r0, _RPS)], acc.at[pl.ds(r0, _RPS)])
        plsc.subcore_barrier()

        fire_idx(0, 0)
        fire_idx(1, 1)
        fire_idx(2, 2)
        wait_idx(0)
        fire_gathers(0)
        fire_idx(3, 3)
        wait_idx(1)
        fire_gathers(1)
        wait_gathers(0)
        fire_scatters(0)

        def blk(i0, carry):
            for r in range(_RD):
                g = 2 + i0 * _RD + r
                wait_scatters(r)
                fire_idx(g + 2, r)
                wait_idx((2 + r) % _RD)
                fire_gathers((2 + r) % _RD)
                wait_gathers((1 + r) % _RD)
                fire_scatters((1 + r) % _RD)
            return carry

        lax.fori_loop(0, nblk, blk, 0)

        wait_scatters(0)
        wait_idx(2)
        fire_gathers(2)
        wait_gathers(1)
        fire_scatters(1)
        wait_scatters(1)
        wait_idx(3)
        fire_gathers(3)
        wait_gathers(2)
        fire_scatters(2)
        wait_gathers(3)
        fire_scatters(3)
        wait_scatters(2)
        wait_scatters(3)

        plsc.subcore_barrier()
        pltpu.sync_copy(acc.at[pl.ds(r0, _RPS)], out_h.at[c, pl.ds(r0, _RPS)])

    return pl.kernel(
        body,
        out_type=jax.ShapeDtypeStruct((_NC, _NACC, _D), jnp.float32),
        mesh=mesh,
        scratch_types=[
            pltpu.VMEM_SHARED((_NACC, _D), jnp.float32),
            [pltpu.VMEM((k, _SUB), jnp.int32) for _ in range(_RD)],
            [pltpu.VMEM((k, _SUB), jnp.int32) for _ in range(_RD)],
            [pltpu.VMEM((k, _SUB, _D), jnp.float32) for _ in range(_RD)],
            [pltpu.SemaphoreType.DMA for _ in range(_RD)],
            [pltpu.SemaphoreType.DMA for _ in range(_RD)],
            [pltpu.SemaphoreType.DMA for _ in range(_RD)],
        ],
        compiler_params=_SC_PARAMS,
    )


def _count_kernel(epw0, epw1):
    k = _KC
    n0 = epw0 // (k * _SUB)
    n1 = epw1 // (k * _SUB)
    assert n0 % _RD == 0 and n1 % _RD == 0 and min(n0, n1) >= 2 * _RD
    mesh = plsc.VectorSubcoreMesh(core_axis_name="c", subcore_axis_name="s")

    def body(ei_h, z1_h, ones_h, cnt_h, cacc, dst_v, ones_v, isem, ssem):
        c = lax.axis_index("c")
        s = lax.axis_index("s")
        r0 = s * _RPS
        rpw = jnp.where(c == 0, epw0 // _SUB, epw1 // _SUB)
        wrow = c * (_NS * (epw0 // _SUB)) + s * rpw
        nblk = jnp.where(c == 0, (n0 - 4) // _RD, (n1 - 4) // _RD)

        def fire_idx(g, b):
            pltpu.async_copy(ei_h.at[1, pl.ds(wrow + g * k, k)], dst_v[b],
                             isem[b])

        def wait_idx(b):
            pltpu.make_async_copy(ei_h.at[1, pl.ds(0, k)], dst_v[b],
                                  isem[b]).wait()

        def fire_scatters(b):
            for j in range(k):
                pltpu.async_copy(ones_v.at[0], cacc.at[dst_v[b].at[j]],
                                 ssem[b], add=True)

        def wait_scatters(b):
            for j in range(k):
                pltpu.make_async_copy(ones_v.at[0], cacc.at[pl.ds(0, _SUB)],
                                      ssem[b]).wait()

        pltpu.sync_copy(z1_h.at[pl.ds(r0, _RPS)], cacc.at[pl.ds(r0, _RPS)])
        pltpu.sync_copy(ones_h, ones_v)
        plsc.subcore_barrier()

        fire_idx(0, 0)
        fire_idx(1, 1)
        fire_idx(2, 2)
        wait_idx(0)
        fire_scatters(0)
        fire_idx(3, 3)
        wait_idx(1)
        fire_scatters(1)

        def blk(i0, carry):
            for r in range(_RD):
                g = 2 + i0 * _RD + r
                wait_scatters(r)
                fire_idx(g + 2, r)
                wait_idx((2 + r) % _RD)
                fire_scatters((2 + r) % _RD)
            return carry

        lax.fori_loop(0, nblk, blk, 0)

        wait_scatters(0)
        wait_idx(2)
        fire_scatters(2)
        wait_scatters(1)
        wait_idx(3)
        fire_scatters(3)
        wait_scatters(2)
        wait_scatters(3)

        plsc.subcore_barrier()
        pltpu.sync_copy(cacc.at[pl.ds(r0, _RPS)], cnt_h.at[c, pl.ds(r0, _RPS)])

    return pl.kernel(
        body,
        out_type=jax.ShapeDtypeStruct((_NC, _NACC), jnp.float32),
        mesh=mesh,
        scratch_types=[
            pltpu.VMEM_SHARED((_NACC,), jnp.float32),
            [pltpu.VMEM((k, _SUB), jnp.int32) for _ in range(_RD)],
            pltpu.VMEM((1, _SUB), jnp.float32),
            [pltpu.SemaphoreType.DMA for _ in range(_RD)],
            [pltpu.SemaphoreType.DMA for _ in range(_RD)],
        ],
        compiler_params=_SC_PARAMS,
    )


def _dense_body(p_ref, x_ref, c8_ref, s_ref, wl_ref, wr_ref, b_ref, o_ref):
    hi = jax.lax.Precision.HIGHEST
    cexp = jnp.dot(c8_ref[0] + c8_ref[1], s_ref[...], precision=hi,
                   preferred_element_type=jnp.float32)
    agg = (p_ref[0] + p_ref[1]) / jnp.maximum(cexp, 1.0)
    o_ref[...] = (
        jnp.dot(agg, wl_ref[...], precision=hi, preferred_element_type=jnp.float32)
        + jnp.dot(x_ref[...], wr_ref[...], precision=hi,
                  preferred_element_type=jnp.float32)
        + b_ref[...]
    )


_BR = 1568


def _dense(p, xr, c8, smat, wlb, wrb, bias):
    grid = (_ROWS // _BR,)
    return pl.pallas_call(
        _dense_body,
        grid=grid,
        in_specs=[
            pl.BlockSpec((_NC, _BR, 128), lambda i: (0, i, 0)),
            pl.BlockSpec((_BR, 128), lambda i: (i, 0)),
            pl.BlockSpec((_NC, _BR, 8), lambda i: (0, i, 0)),
            pl.BlockSpec((8, 128), lambda i: (0, 0)),
            pl.BlockSpec((128, 128), lambda i: (0, 0)),
            pl.BlockSpec((128, 128), lambda i: (0, 0)),
            pl.BlockSpec((1, 128), lambda i: (0, 0)),
        ],
        out_specs=pl.BlockSpec((_BR, 128), lambda i: (i, 0)),
        out_shape=jax.ShapeDtypeStruct((_ROWS, 128), jnp.float32),
    )(p, xr, c8, smat, wlb, wrb, bias)


def kernel(x, edge_index, Wl0, bl0, Wr0, Wlin0, blin0, Wl1, bl1, Wr1, Wlin1,
           blin1, Wl2, bl2, Wr2, Wlin2, blin2):
    ei = edge_index.astype(jnp.int32)
    e = ei.shape[1]
    epw0, epw1 = _split_epw(e)
    pad = jnp.broadcast_to(jnp.array([[0], [_N]], jnp.int32),
                           (2, _NS * (epw0 + epw1) - e))
    ei2 = jnp.concatenate([ei, pad], axis=1).reshape(2, -1, _SUB)

    z2 = jnp.zeros((_NACC, _D), jnp.float32)
    z1 = jnp.zeros((_NACC,), jnp.float32)
    ones = jnp.ones((1, _SUB), jnp.float32)

    eye8 = jnp.eye(8, dtype=jnp.float32)
    smat_np = np.zeros((8, 128), np.float32)
    for i in range(8):
        smat_np[i, 16 * i:16 * (i + 1)] = 1.0
    smat = jnp.asarray(smat_np)

    params = [(Wl0, bl0, Wr0, Wlin0, blin0), (Wl1, bl1, Wr1, Wlin1, blin1),
              (Wl2, bl2, Wr2, Wlin2, blin2)]

    sc_sum = _sum_kernel(epw0, epw1)
    cnt = _count_kernel(epw0, epw1)(ei2, z1, ones)
    c8 = cnt.reshape(_NC, _ROWS, 8)

    xl = jnp.concatenate([x, jnp.zeros((_NACC - _N, _D), jnp.float32)])
    for (Wl, bl, Wr, Wlin, blin) in params:
        p = sc_sum(xl, ei2, z2)
        wlb = jnp.kron(eye8, Wl)
        wrb = jnp.kron(eye8, Wr + Wlin)
        bias = jnp.tile(bl + blin, 8).reshape(1, 128)
        xp = _dense(p.reshape(_NC, _ROWS, 128), xl.reshape(_ROWS, 128), c8,
                    smat, wlb, wrb, bias)
        xl = xp.reshape(_NACC, _D)
    return xl[:_N]


# speedup vs baseline: 1.5354x; 1.0011x over previous
import functools

import numpy as np
import jax
import jax.numpy as jnp
from jax import lax
from jax.experimental import pallas as pl
from jax.experimental.pallas import tpu as pltpu
from jax.experimental.pallas import tpu_sc as plsc

_N = 100000
_D = 16
_NC = 2
_NS = 16
_NW = _NC * _NS
_SUB = 128
_RD = 4
_KS = 3
_KC = 6
_RPS = 6272
_NACC = _RPS * _NS
_ROWS = _NACC * _D // 128

_FRAC0 = 0.655
_QUANT = 3072

_SC_PARAMS = pltpu.CompilerParams(use_tc_tiling_on_sc=False)


def _split_epw(e):
    epw0 = max(2 * _QUANT, -(-int(_FRAC0 * e / _NS) // _QUANT) * _QUANT)
    rest = max(0, e - _NS * epw0)
    epw1 = max(2 * _QUANT, -(-rest // (_NS * _QUANT)) * _QUANT)
    return epw0, epw1


def _sum_kernel(epw0, epw1):
    k = _KS
    n0 = epw0 // (k * _SUB)
    n1 = epw1 // (k * _SUB)
    assert n0 % _RD == 0 and n1 % _RD == 0 and min(n0, n1) >= 2 * _RD
    mesh = plsc.VectorSubcoreMesh(core_axis_name="c", subcore_axis_name="s")

    ch = k * _SUB

    def body(x_h, ei_h, z2_h, out_h, acc, src_v, dst_v, rows_v,
             isem, gsem, ssem):
        c = lax.axis_index("c")
        s = lax.axis_index("s")
        r0 = s * _RPS
        epw_sel = jnp.where(c == 0, epw0, epw1)
        woff = c * (_NS * epw0) + s * epw_sel
        nblk = jnp.where(c == 0, (n0 - 4) // _RD, (n1 - 4) // _RD)

        def fire_idx(g, b):
            pltpu.async_copy(ei_h.at[0, pl.ds(woff + g * ch, ch)], src_v[b],
                             isem[b])
            pltpu.async_copy(ei_h.at[1, pl.ds(woff + g * ch, ch)], dst_v[b],
                             isem[b])

        def wait_idx(b):
            pltpu.make_async_copy(ei_h.at[0, pl.ds(0, ch)], src_v[b],
                                  isem[b]).wait()
            pltpu.make_async_copy(ei_h.at[0, pl.ds(0, ch)], dst_v[b],
                                  isem[b]).wait()

        def fire_gathers(b):
            pltpu.async_copy(x_h.at[src_v[b]], rows_v[b], gsem[b])

        def wait_gathers(b):
            pltpu.make_async_copy(x_h.at[pl.ds(0, ch)], rows_v[b],
                                  gsem[b]).wait()

        def fire_scatters(b):
            pltpu.async_copy(rows_v[b], acc.at[dst_v[b]], ssem[b], add=True)

        def wait_scatters(b):
            pltpu.make_async_copy(rows_v[b], acc.at[pl.ds(0, ch)],
                                  ssem[b]).wait()

        pltpu.sync_copy(z2_h.at[pl.ds(r0, _RPS)], acc.at[pl.ds(r0, _RPS)])
        plsc.subcore_barrier()

        fire_idx(0, 0)
        fire_idx(1, 1)
        fire_idx(2, 2)
        wait_idx(0)
        fire_gathers(0)
        fire_idx(3, 3)
        wait_idx(1)
        fire_gathers(1)
        wait_gathers(0)
        fire_scatters(0)

        def blk(i0, carry):
            for r in range(_RD):
                g = 2 + i0 * _RD + r
                wait_scatters(r)
                fire_idx(g + 2, r)
                wait_idx((2 + r) % _RD)
                fire_gathers((2 + r) % _RD)
                wait_gathers((1 + r) % _RD)
                fire_scatters((1 + r) % _RD)
            return carry

        lax.fori_loop(0, nblk, blk, 0)

        wait_scatters(0)
        wait_idx(2)
        fire_gathers(2)
        wait_gathers(1)
        fire_scatters(1)
        wait_scatters(1)
        wait_idx(3)
        fire_gathers(3)
        wait_gathers(2)
        fire_scatters(2)
        wait_gathers(3)
        fire_scatters(3)
        wait_scatters(2)
        wait_scatters(3)

        plsc.subcore_barrier()
        pltpu.sync_copy(acc.at[pl.ds(r0, _RPS)], out_h.at[c, pl.ds(r0, _RPS)])

    return pl.kernel(
        body,
        out_type=jax.ShapeDtypeStruct((_NC, _NACC, _D), jnp.float32),
        mesh=mesh,
        scratch_types=[
            pltpu.VMEM_SHARED((_NACC, _D), jnp.float32),
            [pltpu.VMEM((k * _SUB,), jnp.int32) for _ in range(_RD)],
            [pltpu.VMEM((k * _SUB,), jnp.int32) for _ in range(_RD)],
            [pltpu.VMEM((k * _SUB, _D), jnp.float32) for _ in range(_RD)],
            [pltpu.SemaphoreType.DMA for _ in range(_RD)],
            [pltpu.SemaphoreType.DMA for _ in range(_RD)],
            [pltpu.SemaphoreType.DMA for _ in range(_RD)],
        ],
        compiler_params=_SC_PARAMS,
    )


def _count_kernel(epw0, epw1):
    k = _KC
    n0 = epw0 // (k * _SUB)
    n1 = epw1 // (k * _SUB)
    assert n0 % _RD == 0 and n1 % _RD == 0 and min(n0, n1) >= 2 * _RD
    mesh = plsc.VectorSubcoreMesh(core_axis_name="c", subcore_axis_name="s")

    ch = k * _SUB

    def body(ei_h, z1_h, ones_h, cnt_h, cacc, dst_v, ones_v, isem, ssem):
        c = lax.axis_index("c")
        s = lax.axis_index("s")
        r0 = s * _RPS
        epw_sel = jnp.where(c == 0, epw0, epw1)
        woff = c * (_NS * epw0) + s * epw_sel
        nblk = jnp.where(c == 0, (n0 - 4) // _RD, (n1 - 4) // _RD)

        def fire_idx(g, b):
            pltpu.async_copy(ei_h.at[1, pl.ds(woff + g * ch, ch)], dst_v[b],
                             isem[b])

        def wait_idx(b):
            pltpu.make_async_copy(ei_h.at[1, pl.ds(0, ch)], dst_v[b],
                                  isem[b]).wait()

        def fire_scatters(b):
            pltpu.async_copy(ones_v, cacc.at[dst_v[b]], ssem[b], add=True)

        def wait_scatters(b):
            pltpu.make_async_copy(ones_v, cacc.at[pl.ds(0, ch)],
                                  ssem[b]).wait()

        pltpu.sync_copy(z1_h.at[pl.ds(r0, _RPS)], cacc.at[pl.ds(r0, _RPS)])
        pltpu.sync_copy(ones_h, ones_v)
        plsc.subcore_barrier()

        fire_idx(0, 0)
        fire_idx(1, 1)
        fire_idx(2, 2)
        wait_idx(0)
        fire_scatters(0)
        fire_idx(3, 3)
        wait_idx(1)
        fire_scatters(1)

        def blk(i0, carry):
            for r in range(_RD):
                g = 2 + i0 * _RD + r
                wait_scatters(r)
                fire_idx(g + 2, r)
                wait_idx((2 + r) % _RD)
                fire_scatters((2 + r) % _RD)
            return carry

        lax.fori_loop(0, nblk, blk, 0)

        wait_scatters(0)
        wait_idx(2)
        fire_scatters(2)
        wait_scatters(1)
        wait_idx(3)
        fire_scatters(3)
        wait_scatters(2)
        wait_scatters(3)

        plsc.subcore_barrier()
        pltpu.sync_copy(cacc.at[pl.ds(r0, _RPS)], cnt_h.at[c, pl.ds(r0, _RPS)])

    return pl.kernel(
        body,
        out_type=jax.ShapeDtypeStruct((_NC, _NACC), jnp.float32),
        mesh=mesh,
        scratch_types=[
            pltpu.VMEM_SHARED((_NACC,), jnp.float32),
            [pltpu.VMEM((k * _SUB,), jnp.int32) for _ in range(_RD)],
            pltpu.VMEM((k * _SUB,), jnp.float32),
            [pltpu.SemaphoreType.DMA for _ in range(_RD)],
            [pltpu.SemaphoreType.DMA for _ in range(_RD)],
        ],
        compiler_params=_SC_PARAMS,
    )


def _dense_body(p_ref, x_ref, c8_ref, s_ref, wl_ref, wr_ref, b_ref, o_ref):
    hi = jax.lax.Precision.HIGHEST
    cexp = jnp.dot(c8_ref[0] + c8_ref[1], s_ref[...], precision=hi,
                   preferred_element_type=jnp.float32)
    agg = (p_ref[0] + p_ref[1]) / jnp.maximum(cexp, 1.0)
    o_ref[...] = (
        jnp.dot(agg, wl_ref[...], precision=hi, preferred_element_type=jnp.float32)
        + jnp.dot(x_ref[...], wr_ref[...], precision=hi,
                  preferred_element_type=jnp.float32)
        + b_ref[...]
    )


_BR = 1568


def _dense(p, xr, c8, smat, wlb, wrb, bias):
    grid = (_ROWS // _BR,)
    return pl.pallas_call(
        _dense_body,
        grid=grid,
        in_specs=[
            pl.BlockSpec((_NC, _BR, 128), lambda i: (0, i, 0)),
            pl.BlockSpec((_BR, 128), lambda i: (i, 0)),
            pl.BlockSpec((_NC, _BR, 8), lambda i: (0, i, 0)),
            pl.BlockSpec((8, 128), lambda i: (0, 0)),
            pl.BlockSpec((128, 128), lambda i: (0, 0)),
            pl.BlockSpec((128, 128), lambda i: (0, 0)),
            pl.BlockSpec((1, 128), lambda i: (0, 0)),
        ],
        out_specs=pl.BlockSpec((_BR, 128), lambda i: (i, 0)),
        out_shape=jax.ShapeDtypeStruct((_ROWS, 128), jnp.float32),
    )(p, xr, c8, smat, wlb, wrb, bias)


def kernel(x, edge_index, Wl0, bl0, Wr0, Wlin0, blin0, Wl1, bl1, Wr1, Wlin1,
           blin1, Wl2, bl2, Wr2, Wlin2, blin2):
    ei = edge_index.astype(jnp.int32)
    e = ei.shape[1]
    epw0, epw1 = _split_epw(e)
    pad = jnp.broadcast_to(jnp.array([[0], [_N]], jnp.int32),
                           (2, _NS * (epw0 + epw1) - e))
    ei2 = jnp.concatenate([ei, pad], axis=1)

    z2 = jnp.zeros((_NACC, _D), jnp.float32)
    z1 = jnp.zeros((_NACC,), jnp.float32)
    ones = jnp.ones((_KC * _SUB,), jnp.float32)

    eye8 = jnp.eye(8, dtype=jnp.float32)
    smat_np = np.zeros((8, 128), np.float32)
    for i in range(8):
        smat_np[i, 16 * i:16 * (i + 1)] = 1.0
    smat = jnp.asarray(smat_np)

    params = [(Wl0, bl0, Wr0, Wlin0, blin0), (Wl1, bl1, Wr1, Wlin1, blin1),
              (Wl2, bl2, Wr2, Wlin2, blin2)]

    sc_sum = _sum_kernel(epw0, epw1)
    cnt = _count_kernel(epw0, epw1)(ei2, z1, ones)
    c8 = cnt.reshape(_NC, _ROWS, 8)

    xl = jnp.concatenate([x, jnp.zeros((_NACC - _N, _D), jnp.float32)])
    for (Wl, bl, Wr, Wlin, blin) in params:
        p = sc_sum(xl, ei2, z2)
        wlb = jnp.kron(eye8, Wl)
        wrb = jnp.kron(eye8, Wr + Wlin)
        bias = jnp.tile(bl + blin, 8).reshape(1, 128)
        xp = _dense(p.reshape(_NC, _ROWS, 128), xl.reshape(_ROWS, 128), c8,
                    smat, wlb, wrb, bias)
        xl = xp.reshape(_NACC, _D)
    return xl[:_N]
